# baseline (speedup 1.0000x reference)
import functools

import jax
import jax.numpy as jnp
from jax import lax
from jax.experimental import pallas as pl
from jax.experimental.pallas import tpu as pltpu
from jax.experimental.pallas import tpu_sc as plsc

F32 = jnp.float32

_NB = 1000
_EB = 4000
_ACCW = 144
_K = 16


def _recip(z):
    r = lax.bitcast_convert_type(
        jnp.int32(0x7EF07EBB) - lax.bitcast_convert_type(z, jnp.int32), F32)
    for _ in range(3):
        r = r * (2.0 - z * r)
    return r


def _silu(v):
    return v * _recip(1.0 + jnp.exp(-jnp.maximum(v, -80.0)))


_GDN = lax.GatherDimensionNumbers(
    offset_dims=(), collapsed_slice_dims=(0,), start_index_map=(0,))


def _lane_shuffle(v, idx):
    return lax.gather(v, idx[:, None], _GDN, slice_sizes=(1,),
                      mode=lax.GatherScatterMode.PROMISE_IN_BOUNDS)


def _bcast_sum(v, lane):
    for k in (1, 2, 4, 8):
        v = v + _lane_shuffle(v, lane ^ k)
    return v


def _prep_node_body(h_ref, x_ref, ws_ref, wd_ref, bd_ref, s_ref, d_ref):
    hb = h_ref[...]
    xb = x_ref[...]
    pad = jnp.zeros((xb.shape[0], 13), F32)
    s = jnp.dot(hb, ws_ref[...], preferred_element_type=F32)
    d = jnp.dot(hb, wd_ref[...], preferred_element_type=F32) + bd_ref[...]
    s_ref[...] = jnp.concatenate([s, xb, pad], axis=1)
    d_ref[...] = jnp.concatenate([d, xb, pad], axis=1)


def _prep_edge_body(ed_ref, we1_ref, be1_ref, bm_ref, bb_ref, p_ref):
    t1 = ed_ref[...] * we1_ref[...] + be1_ref[...]
    p_ref[...] = jnp.dot(_silu(t1), bm_ref[...],
                         preferred_element_type=F32) + bb_ref[...]


def _combine_body(h_ref, x_ref, acc_ref, wn2_ref, bn2_ref, ho_ref, xo_ref):
    acc = jnp.sum(acc_ref[...], axis=0)
    sagg = acc[:, 0:128]
    xagg = acc[:, 128:131]
    deg = acc[:, 131:132]
    ho_ref[...] = (h_ref[...]
                   + jnp.dot(sagg, wn2_ref[...], preferred_element_type=F32)
                   + deg * bn2_ref[...])
    xo_ref[...] = x_ref[...] + xagg


_INTERP = False


def _make_sc_kernel(N, E, NC, NS):
    NW = NC * NS
    epw = E // NW
    nchunk = epw // _K
    assert nchunk % 2 == 1 and nchunk >= 3
    npairs = (nchunk - 1) // 2
    zr = _K
    npad = -(-N // (NS * zr)) * (NS * zr)
    rows_per_tile = npad // NS
    mesh = plsc.VectorSubcoreMesh(core_axis_name="c", subcore_axis_name="s",
                                  num_cores=NC)

    @functools.partial(
        pl.kernel,
        out_type=jax.ShapeDtypeStruct((NC, npad, _ACCW), F32),
        mesh=mesh,
        scratch_types=[
            pltpu.VMEM((_K,), jnp.int32),
            pltpu.VMEM((_K,), jnp.int32),
            pltpu.VMEM((_K,), jnp.int32),
            pltpu.VMEM((_K,), jnp.int32),
            pltpu.VMEM((_K, 272), F32),
            pltpu.VMEM((_K, 272), F32),
            pltpu.VMEM((_K, 272), F32),
            pltpu.VMEM((_K, 272), F32),
            pltpu.VMEM((_K, 256), F32),
            pltpu.VMEM((_K, 256), F32),
            pltpu.VMEM((_K, _ACCW), F32),
            pltpu.VMEM((128,), F32),
            pltpu.VMEM_SHARED((npad, _ACCW), F32),
            pltpu.SemaphoreType.DMA,
            pltpu.SemaphoreType.DMA,
            pltpu.SemaphoreType.DMA,
            pltpu.SemaphoreType.DMA,
        ],
        compiler_params=pltpu.CompilerParams(use_tc_tiling_on_sc=False),
        interpret=_INTERP,
    )
    def sc_fn(s_hbm, d_hbm, p_hbm, src_hbm, dst_hbm, wc2_hbm,
              out_hbm, sidx0, sidx1, didx0, didx1, srows0, srows1,
              drows0, drows1, prows0, prows1, orows, wc2v, acc,
              semld0, semld1, semidx0, semidx1):
        c = lax.axis_index("c")
        s = lax.axis_index("s")
        wid = s * NC + c
        zero16 = jnp.zeros((16,), F32)
        slots = ((sidx0, didx0, srows0, drows0, prows0, semld0, semidx0),
                 (sidx1, didx1, srows1, drows1, prows1, semld1, semidx1))

        def _zrow(r, carry):
            for j in range(_ACCW // 16):
                orows[r, pl.ds(j * 16, 16)] = zero16
            return carry
        lax.fori_loop(0, zr, _zrow, 0)
        row0 = s * rows_per_tile
        for j in range(rows_per_tile // zr):
            pltpu.sync_copy(orows, acc.at[pl.ds(row0 + j * zr, zr)])
        pltpu.sync_copy(wc2_hbm, wc2v)
        plsc.subcore_barrier()

        lane = lax.broadcasted_iota(jnp.int32, (16,), 0)
        onehot3 = jnp.where(lane == 3, 1.0, 0.0).astype(F32)
        rot1 = jnp.where(lane < 3, lax.rem(lane + 1, 3), 3)
        rot2 = jnp.where(lane < 3, lax.rem(lane + 2, 3), 3)
        ebase = wid * epw

        def _base(ci):
            return pl.multiple_of(ebase + ci * _K, 8)

        def _idx_issue(ci, sl):
            sidx, didx, _, _, _, _, semidx = sl
            base = _base(ci)
            pltpu.async_copy(src_hbm.at[pl.ds(base, _K)], sidx, semidx)
            pltpu.async_copy(dst_hbm.at[pl.ds(base, _K)], didx, semidx)

        def _idx_wait(sl):
            sidx, didx, _, _, _, _, semidx = sl
            pltpu.make_async_copy(src_hbm.at[pl.ds(0, _K)], sidx, semidx).wait()
            pltpu.make_async_copy(dst_hbm.at[pl.ds(0, _K)], didx, semidx).wait()

        def _ld_issue(ci, sl):
            sidx, didx, srows, drows, prows, semld, _ = sl
            base = _base(ci)
            pltpu.async_copy(s_hbm.at[sidx], srows, semld)
            pltpu.async_copy(d_hbm.at[didx], drows, semld)
            pltpu.async_copy(p_hbm.at[pl.ds(base, _K)], prows, semld)

        def _ld_wait(sl):
            sidx, didx, srows, drows, prows, semld, _ = sl
            pltpu.make_async_copy(s_hbm.at[sidx], srows, semld).wait()
            pltpu.make_async_copy(d_hbm.at[didx], drows, semld).wait()
            pltpu.make_async_copy(p_hbm.at[pl.ds(0, _K)], prows, semld).wait()

        def _compute_scatter(sl):
            _, didx, srows, drows, prows, _, _ = sl

            def _edge4(i4, ecarry):
                for u in range(4):
                    i = i4 * 4 + u
                    for j in range(8):
                        a = (srows[i, pl.ds(j * 16, 16)]
                             + drows[i, pl.ds(j * 16, 16)]
                             + prows[i, pl.ds(j * 16, 16)])
                        orows[i, pl.ds(j * 16, 16)] = _silu(a)
                    dot = jnp.zeros((16,), F32)
                    for j in range(8):
                        a = (srows[i, pl.ds(128 + j * 16, 16)]
                             + drows[i, pl.ds(128 + j * 16, 16)]
                             + prows[i, pl.ds(128 + j * 16, 16)])
                        dot = dot + _silu(a) * wc2v[pl.ds(j * 16, 16)]
                    w = _bcast_sum(dot, lane)
                    dv = srows[i, pl.ds(256, 16)] - drows[i, pl.ds(256, 16)]
                    d2 = dv * dv
                    n2 = d2 + _lane_shuffle(d2, rot1) + _lane_shuffle(d2, rot2)
                    n2c = jnp.maximum(n2, 1e-30)
                    y = lax.bitcast_convert_type(
                        jnp.int32(0x5F3759DF)
                        - (lax.bitcast_convert_type(n2c, jnp.int32) >> 1),
                        F32)
                    for _ in range(3):
                        y = y * (1.5 - 0.5 * n2c * y * y)
                    inv_len = _recip(jnp.maximum(n2c * y, 1e-8))
                    orows[i, pl.ds(128, 16)] = dv * inv_len * w + onehot3
                return ecarry
            lax.fori_loop(0, _K // 4, _edge4, 0)
            pltpu.sync_copy(orows, acc.at[didx], add=True)

        pltpu.sync_copy(src_hbm.at[pl.ds(_base(0), _K)], sidx0)
        pltpu.sync_copy(dst_hbm.at[pl.ds(_base(0), _K)], didx0)
        _ld_issue(0, slots[0])
        _idx_issue(1, slots[1])

        def _pair(g, carry):
            i0 = 2 * g
            _ld_wait(slots[0])
            _idx_wait(slots[1])
            _ld_issue(i0 + 1, slots[1])
            _compute_scatter(slots[0])
            _idx_issue(i0 + 2, slots[0])
            _ld_wait(slots[1])
            _idx_wait(slots[0])
            _ld_issue(i0 + 2, slots[0])
            _compute_scatter(slots[1])

            @pl.when(i0 + 3 < nchunk)
            def _():
                _idx_issue(i0 + 3, slots[1])
            return carry
        lax.fori_loop(0, npairs, _pair, 0)
        _ld_wait(slots[0])
        _compute_scatter(slots[0])
        plsc.subcore_barrier()

        for j in range(rows_per_tile // zr):
            r = row0 + j * zr
            pltpu.sync_copy(acc.at[pl.ds(r, zr)], orows)
            pltpu.sync_copy(orows, out_hbm.at[c, pl.ds(r, zr)])

    return sc_fn


def kernel(h, x, edge_index, edge_dist, We1, be1, We2, be2,
           Wn1, bn1, Wn2, bn2, Wc1, bc1, Wc2):
    N, ND = h.shape
    E = edge_dist.shape[0]
    XD = x.shape[1]
    info = plsc.get_sparse_core_info()
    NC, NS = info.num_cores, info.num_subcores
    assert ND == 128 and N % _NB == 0 and E % _EB == 0
    assert E % (NC * NS * _K) == 0

    WS = jnp.concatenate([Wn1[:ND], Wc1[:ND]], axis=1)
    WD = jnp.concatenate([Wn1[ND:2 * ND], Wc1[ND:2 * ND]], axis=1)
    bD = jnp.concatenate([bn1, bc1])[None, :]
    M = jnp.concatenate([Wn1[2 * ND:], Wc1[2 * ND:]], axis=1)
    BM = We2 @ M
    bb = (be2 @ M)[None, :] + bD

    nb = N // _NB
    S, D = pl.pallas_call(
        _prep_node_body,
        grid=(nb,),
        in_specs=[
            pl.BlockSpec((_NB, ND), lambda i: (i, 0)),
            pl.BlockSpec((_NB, XD), lambda i: (i, 0)),
            pl.BlockSpec((ND, 256), lambda i: (0, 0)),
            pl.BlockSpec((ND, 256), lambda i: (0, 0)),
            pl.BlockSpec((1, 256), lambda i: (0, 0)),
        ],
        out_specs=[
            pl.BlockSpec((_NB, 272), lambda i: (i, 0)),
            pl.BlockSpec((_NB, 272), lambda i: (i, 0)),
        ],
        out_shape=[
            jax.ShapeDtypeStruct((N, 272), F32),
            jax.ShapeDtypeStruct((N, 272), F32),
        ],
    )(h, x, WS, WD, bD)

    P = pl.pallas_call(
        _prep_edge_body,
        grid=(E // _EB,),
        in_specs=[
            pl.BlockSpec((_EB, 1), lambda i: (i, 0)),
            pl.BlockSpec((1, 16), lambda i: (0, 0)),
            pl.BlockSpec((1, 16), lambda i: (0, 0)),
            pl.BlockSpec((16, 256), lambda i: (0, 0)),
            pl.BlockSpec((1, 256), lambda i: (0, 0)),
        ],
        out_specs=pl.BlockSpec((_EB, 256), lambda i: (i, 0)),
        out_shape=jax.ShapeDtypeStruct((E, 256), F32),
    )(edge_dist[:, None], We1, be1[None, :], BM, bb)

    sc_fn = _make_sc_kernel(N, E, NC, NS)
    acc = sc_fn(S, D, P, edge_index[0], edge_index[1], Wc2[:, 0])
    acc = acc[:, :N, :]

    h_out, x_out = pl.pallas_call(
        _combine_body,
        grid=(nb,),
        in_specs=[
            pl.BlockSpec((_NB, ND), lambda i: (i, 0)),
            pl.BlockSpec((_NB, XD), lambda i: (i, 0)),
            pl.BlockSpec((NC, _NB, _ACCW), lambda i: (0, i, 0)),
            pl.BlockSpec((ND, ND), lambda i: (0, 0)),
            pl.BlockSpec((1, ND), lambda i: (0, 0)),
        ],
        out_specs=[
            pl.BlockSpec((_NB, ND), lambda i: (i, 0)),
            pl.BlockSpec((_NB, XD), lambda i: (i, 0)),
        ],
        out_shape=[
            jax.ShapeDtypeStruct((N, ND), F32),
            jax.ShapeDtypeStruct((N, XD), F32),
        ],
    )(h, x, acc, Wn2, bn2[None, :])
    return (h_out, x_out)


# speedup vs baseline: 2.0144x; 2.0144x over previous
import functools

import jax
import jax.numpy as jnp
from jax import lax
from jax.experimental import pallas as pl
from jax.experimental.pallas import tpu as pltpu
from jax.experimental.pallas import tpu_sc as plsc

F32 = jnp.float32

_NB = 1000
_EB = 2000
_ACCW = 144
_KG = 40
_KS = 80


def _silu(v):
    return v / (1.0 + jnp.exp(-v))


def _prep_node_body(h_ref, x_ref, ws_ref, wd_ref, bd_ref, s_ref, d_ref):
    hb = h_ref[...]
    xb = x_ref[...]
    pad = jnp.zeros((xb.shape[0], 13), F32)
    s = jnp.dot(hb, ws_ref[...], preferred_element_type=F32)
    d = jnp.dot(hb, wd_ref[...], preferred_element_type=F32) + bd_ref[...]
    s_ref[...] = jnp.concatenate([s, xb, pad], axis=1)
    d_ref[...] = jnp.concatenate([d, -xb, pad], axis=1)


def _edge_mid_body(pre_ref, ed_ref, we1_ref, be1_ref, bm_ref, bb_ref,
                   wc2_ref, out_ref):
    pre = pre_ref[...]
    t1 = ed_ref[...] * we1_ref[...] + be1_ref[...]
    p = jnp.dot(_silu(t1), bm_ref[...],
                preferred_element_type=F32) + bb_ref[...]
    a = pre[:, 0:256] + p
    sn = _silu(a[:, 0:128])
    sc = _silu(a[:, 128:256])
    w = jnp.dot(sc, wc2_ref[...], preferred_element_type=F32)
    dv = pre[:, 256:272]
    n2 = jnp.sum(dv * dv, axis=1, keepdims=True)
    inv_len = 1.0 / jnp.maximum(jnp.sqrt(n2), 1e-8)
    upd = dv * (w * inv_len)
    lanecol = lax.broadcasted_iota(jnp.int32, (1, 16), 1)
    upd = upd + jnp.where(lanecol == 3, 1.0, 0.0).astype(F32)
    out_ref[...] = jnp.concatenate([sn, upd], axis=1)


def _combine_body(h_ref, x_ref, acc_ref, wn2_ref, bn2_ref, ho_ref, xo_ref):
    acc = jnp.sum(acc_ref[...], axis=0)
    sagg = acc[:, 0:128]
    xagg = acc[:, 128:131]
    deg = acc[:, 131:132]
    ho_ref[...] = (h_ref[...]
                   + jnp.dot(sagg, wn2_ref[...], preferred_element_type=F32)
                   + deg * bn2_ref[...])
    xo_ref[...] = x_ref[...] + xagg


def _make_sc_gather(N, E, NC, NS):
    NW = NC * NS
    epw = E // NW
    nchunk = epw // _KG
    npairs = nchunk // 2
    leftover = nchunk - 2 * npairs
    mesh = plsc.VectorSubcoreMesh(core_axis_name="c", subcore_axis_name="s",
                                  num_cores=NC)

    @functools.partial(
        pl.kernel,
        out_type=jax.ShapeDtypeStruct((E, 272), F32),
        mesh=mesh,
        scratch_types=[
            pltpu.VMEM((_KG,), jnp.int32),
            pltpu.VMEM((_KG,), jnp.int32),
            pltpu.VMEM((_KG,), jnp.int32),
            pltpu.VMEM((_KG,), jnp.int32),
            pltpu.VMEM((_KG, 272), F32),
            pltpu.VMEM((_KG, 272), F32),
            pltpu.VMEM((_KG, 272), F32),
            pltpu.VMEM((_KG, 272), F32),
            pltpu.VMEM((_KG, 272), F32),
            pltpu.SemaphoreType.DMA,
            pltpu.SemaphoreType.DMA,
            pltpu.SemaphoreType.DMA,
            pltpu.SemaphoreType.DMA,
        ],
        compiler_params=pltpu.CompilerParams(use_tc_tiling_on_sc=False),
    )
    def gather_fn(s_hbm, d_hbm, src_hbm, dst_hbm, pre_hbm,
                  sidx0, sidx1, didx0, didx1, srows0, srows1,
                  drows0, drows1, orows, semld0, semld1, semidx0, semidx1):
        c = lax.axis_index("c")
        s = lax.axis_index("s")
        wid = s * NC + c
        ebase = wid * epw
        slots = ((sidx0, didx0, srows0, drows0, semld0, semidx0),
                 (sidx1, didx1, srows1, drows1, semld1, semidx1))

        def _base(ci):
            return pl.multiple_of(ebase + ci * _KG, 8)

        def _idx_issue(ci, sl):
            sidx, didx, _, _, _, semidx = sl
            base = _base(ci)
            pltpu.async_copy(src_hbm.at[pl.ds(base, _KG)], sidx, semidx)
            pltpu.async_copy(dst_hbm.at[pl.ds(base, _KG)], didx, semidx)

        def _idx_wait(sl):
            sidx, didx, _, _, _, semidx = sl
            pltpu.make_async_copy(src_hbm.at[pl.ds(0, _KG)], sidx, semidx).wait()
            pltpu.make_async_copy(dst_hbm.at[pl.ds(0, _KG)], didx, semidx).wait()

        def _ld_issue(sl):
            sidx, didx, srows, drows, semld, _ = sl
            pltpu.async_copy(s_hbm.at[sidx], srows, semld)
            pltpu.async_copy(d_hbm.at[didx], drows, semld)

        def _ld_wait(sl):
            sidx, didx, srows, drows, semld, _ = sl
            pltpu.make_async_copy(s_hbm.at[sidx], srows, semld).wait()
            pltpu.make_async_copy(d_hbm.at[didx], drows, semld).wait()

        def _process(ci, sl):
            _, _, srows, drows, _, _ = sl

            def _edge2(i2, ecarry):
                for u in range(2):
                    i = i2 * 2 + u
                    for j in range(17):
                        orows[i, pl.ds(j * 16, 16)] = (
                            srows[i, pl.ds(j * 16, 16)]
                            + drows[i, pl.ds(j * 16, 16)])
                return ecarry
            lax.fori_loop(0, _KG // 2, _edge2, 0)
            pltpu.sync_copy(orows, pre_hbm.at[pl.ds(_base(ci), _KG)])

        pltpu.sync_copy(src_hbm.at[pl.ds(_base(0), _KG)], sidx0)
        pltpu.sync_copy(dst_hbm.at[pl.ds(_base(0), _KG)], didx0)
        _ld_issue(slots[0])
        if nchunk > 1:
            _idx_issue(1, slots[1])

        def _pair(g, carry):
            c0 = 2 * g
            c1 = c0 + 1
            _ld_wait(slots[0])
            _idx_wait(slots[1])
            _ld_issue(slots[1])
            _process(c0, slots[0])

            @pl.when(c0 + 2 < nchunk)
            def _():
                _idx_issue(c0 + 2, slots[0])
            _ld_wait(slots[1])

            @pl.when(c1 + 1 < nchunk)
            def _():
                _idx_wait(slots[0])
                _ld_issue(slots[0])
            _process(c1, slots[1])

            @pl.when(c1 + 2 < nchunk)
            def _():
                _idx_issue(c1 + 2, slots[1])
            return carry
        lax.fori_loop(0, npairs, _pair, 0)
        if leftover:
            _ld_wait(slots[0])
            _process(nchunk - 1, slots[0])

    return gather_fn


def _make_sc_scatter(N, E, NC, NS):
    NW = NC * NS
    epw = E // NW
    nchunk = epw // _KS
    npairs = nchunk // 2
    leftover = nchunk - 2 * npairs
    zr = 16
    npad = -(-N // (NS * zr)) * (NS * zr)
    rows_per_tile = npad // NS
    mesh = plsc.VectorSubcoreMesh(core_axis_name="c", subcore_axis_name="s",
                                  num_cores=NC)

    @functools.partial(
        pl.kernel,
        out_type=jax.ShapeDtypeStruct((NC, npad, _ACCW), F32),
        mesh=mesh,
        scratch_types=[
            pltpu.VMEM((_KS,), jnp.int32),
            pltpu.VMEM((_KS,), jnp.int32),
            pltpu.VMEM((_KS, _ACCW), F32),
            pltpu.VMEM((_KS, _ACCW), F32),
            pltpu.VMEM((zr, _ACCW), F32),
            pltpu.VMEM_SHARED((npad, _ACCW), F32),
            pltpu.SemaphoreType.DMA,
            pltpu.SemaphoreType.DMA,
            pltpu.SemaphoreType.DMA,
            pltpu.SemaphoreType.DMA,
        ],
        compiler_params=pltpu.CompilerParams(use_tc_tiling_on_sc=False),
    )
    def scatter_fn(pay_hbm, dst_hbm, out_hbm,
                   didx0, didx1, prow0, prow1, bounce, acc,
                   semld0, semld1, semidx0, semidx1):
        c = lax.axis_index("c")
        s = lax.axis_index("s")
        wid = s * NC + c
        ebase = wid * epw
        zero16 = jnp.zeros((16,), F32)
        slots = ((didx0, prow0, semld0, semidx0),
                 (didx1, prow1, semld1, semidx1))

        def _zrow(r, carry):
            for j in range(_ACCW // 16):
                bounce[r, pl.ds(j * 16, 16)] = zero16
            return carry
        lax.fori_loop(0, zr, _zrow, 0)
        row0 = s * rows_per_tile
        for j in range(rows_per_tile // zr):
            pltpu.sync_copy(bounce, acc.at[pl.ds(row0 + j * zr, zr)])
        plsc.subcore_barrier()

        def _base(ci):
            return pl.multiple_of(ebase + ci * _KS, 8)

        def _idx_issue(ci, sl):
            didx, _, _, semidx = sl
            pltpu.async_copy(dst_hbm.at[pl.ds(_base(ci), _KS)], didx, semidx)

        def _idx_wait(sl):
            didx, _, _, semidx = sl
            pltpu.make_async_copy(dst_hbm.at[pl.ds(0, _KS)], didx, semidx).wait()

        def _ld_issue(ci, sl):
            _, prow, semld, _ = sl
            pltpu.async_copy(pay_hbm.at[pl.ds(_base(ci), _KS)], prow, semld)

        def _ld_wait(sl):
            _, prow, semld, _ = sl
            pltpu.make_async_copy(pay_hbm.at[pl.ds(0, _KS)], prow, semld).wait()

        def _process(sl):
            didx, prow, _, _ = sl
            pltpu.sync_copy(prow, acc.at[didx], add=True)

        pltpu.sync_copy(dst_hbm.at[pl.ds(_base(0), _KS)], didx0)
        _ld_issue(0, slots[0])
        if nchunk > 1:
            _idx_issue(1, slots[1])

        def _pair(g, carry):
            c0 = 2 * g
            c1 = c0 + 1
            _ld_wait(slots[0])
            _idx_wait(slots[1])
            _ld_issue(c1, slots[1])
            _process(slots[0])

            @pl.when(c0 + 2 < nchunk)
            def _():
                _idx_issue(c0 + 2, slots[0])
            _ld_wait(slots[1])

            @pl.when(c1 + 1 < nchunk)
            def _():
                _idx_wait(slots[0])
                _ld_issue(c1 + 1, slots[0])
            _process(slots[1])

            @pl.when(c1 + 2 < nchunk)
            def _():
                _idx_issue(c1 + 2, slots[1])
            return carry
        lax.fori_loop(0, npairs, _pair, 0)
        if leftover:
            _ld_wait(slots[0])
            _process(slots[0])
        plsc.subcore_barrier()

        for j in range(rows_per_tile // zr):
            r = row0 + j * zr
            pltpu.sync_copy(acc.at[pl.ds(r, zr)], bounce)
            pltpu.sync_copy(bounce, out_hbm.at[c, pl.ds(r, zr)])

    return scatter_fn


def kernel(h, x, edge_index, edge_dist, We1, be1, We2, be2,
           Wn1, bn1, Wn2, bn2, Wc1, bc1, Wc2):
    N, ND = h.shape
    E = edge_dist.shape[0]
    XD = x.shape[1]
    info = plsc.get_sparse_core_info()
    NC, NS = info.num_cores, info.num_subcores
    assert ND == 128 and N % _NB == 0 and E % _EB == 0
    assert E % (NC * NS * _KG) == 0 and E % (NC * NS * _KS) == 0

    WS = jnp.concatenate([Wn1[:ND], Wc1[:ND]], axis=1)
    WD = jnp.concatenate([Wn1[ND:2 * ND], Wc1[ND:2 * ND]], axis=1)
    bD = jnp.concatenate([bn1, bc1])[None, :]
    M = jnp.concatenate([Wn1[2 * ND:], Wc1[2 * ND:]], axis=1)
    BM = We2 @ M
    bb = (be2 @ M)[None, :] + bD

    nb = N // _NB
    S, D = pl.pallas_call(
        _prep_node_body,
        grid=(nb,),
        in_specs=[
            pl.BlockSpec((_NB, ND), lambda i: (i, 0)),
            pl.BlockSpec((_NB, XD), lambda i: (i, 0)),
            pl.BlockSpec((ND, 256), lambda i: (0, 0)),
            pl.BlockSpec((ND, 256), lambda i: (0, 0)),
            pl.BlockSpec((1, 256), lambda i: (0, 0)),
        ],
        out_specs=[
            pl.BlockSpec((_NB, 272), lambda i: (i, 0)),
            pl.BlockSpec((_NB, 272), lambda i: (i, 0)),
        ],
        out_shape=[
            jax.ShapeDtypeStruct((N, 272), F32),
            jax.ShapeDtypeStruct((N, 272), F32),
        ],
    )(h, x, WS, WD, bD)

    gather_fn = _make_sc_gather(N, E, NC, NS)
    pre = gather_fn(S, D, edge_index[0], edge_index[1])

    payload = pl.pallas_call(
        _edge_mid_body,
        grid=(E // _EB,),
        in_specs=[
            pl.BlockSpec((_EB, 272), lambda i: (i, 0)),
            pl.BlockSpec((_EB, 1), lambda i: (i, 0)),
            pl.BlockSpec((1, 16), lambda i: (0, 0)),
            pl.BlockSpec((1, 16), lambda i: (0, 0)),
            pl.BlockSpec((16, 256), lambda i: (0, 0)),
            pl.BlockSpec((1, 256), lambda i: (0, 0)),
            pl.BlockSpec((ND, 1), lambda i: (0, 0)),
        ],
        out_specs=pl.BlockSpec((_EB, _ACCW), lambda i: (i, 0)),
        out_shape=jax.ShapeDtypeStruct((E, _ACCW), F32),
    )(pre, edge_dist[:, None], We1, be1[None, :], BM, bb, Wc2)

    scatter_fn = _make_sc_scatter(N, E, NC, NS)
    acc = scatter_fn(payload, edge_index[1])
    acc = acc[:, :N, :]

    h_out, x_out = pl.pallas_call(
        _combine_body,
        grid=(nb,),
        in_specs=[
            pl.BlockSpec((_NB, ND), lambda i: (i, 0)),
            pl.BlockSpec((_NB, XD), lambda i: (i, 0)),
            pl.BlockSpec((NC, _NB, _ACCW), lambda i: (0, i, 0)),
            pl.BlockSpec((ND, ND), lambda i: (0, 0)),
            pl.BlockSpec((1, ND), lambda i: (0, 0)),
        ],
        out_specs=[
            pl.BlockSpec((_NB, ND), lambda i: (i, 0)),
            pl.BlockSpec((_NB, XD), lambda i: (i, 0)),
        ],
        out_shape=[
            jax.ShapeDtypeStruct((N, ND), F32),
            jax.ShapeDtypeStruct((N, XD), F32),
        ],
    )(h, x, acc, Wn2, bn2[None, :])
    return (h_out, x_out)
